# baseline (speedup 1.0000x reference)
import jax
import jax.numpy as jnp
from jax import lax
from jax.experimental import pallas as pl

_CH = 384
_B = 32
_MAXF_BITS = 0x7F7FFFFF


def _max_kernel(x_ref, m_ref):
    x2 = x_ref[...]
    for k in range(4):
        m_ref[0, k:k + 1] = jnp.max(x2[k * 3136:(k + 1) * 3136, :], axis=0,
                                    keepdims=True)


def _mid_kernel(m_ref, s_ref):
    m = m_ref[...]
    alpha = jax.nn.sigmoid(m)
    beta = 1.0 - alpha

    t = jnp.floor(jnp.sum(jnp.exp(beta - alpha)) / _B).astype(jnp.int32)
    t = jnp.where(t <= 2, 3, t)
    t = jnp.minimum(t, _CH)

    hi = lax.Precision.HIGHEST
    ones_col = jnp.ones((_B, 1), jnp.float32)
    ones_row = jnp.ones((1, _CH), jnp.float32)
    a2_row = jnp.sum(alpha * alpha, axis=0, keepdims=True)
    b2_col = lax.dot_general(beta * beta, ones_col, (((0,), (0,)), ((), ())),
                             precision=hi)
    gt = lax.dot_general(beta, alpha, (((0,), (0,)), ((), ())),
                         precision=hi)
    dt = jnp.sqrt(b2_col + 2.0 * gt + a2_row)
    dti = lax.bitcast_convert_type(dt, jnp.int32)

    def body(_, c):
        lo2, hi2, lot, hit = c
        mid2 = lo2 + lax.shift_right_logical(hi2 - lo2, 1)
        midt = lot + lax.shift_right_logical(hit - lot, 1)
        cnt2 = jnp.sum((dti <= mid2).astype(jnp.int32), axis=0, keepdims=True)
        cntt = jnp.sum((dti <= midt).astype(jnp.int32), axis=0, keepdims=True)
        ok2 = cnt2 >= 2
        okt = cntt >= t
        return (jnp.where(ok2, lo2, mid2 + 1), jnp.where(ok2, mid2, hi2),
                jnp.where(okt, lot, midt + 1), jnp.where(okt, midt, hit))

    lo0 = jnp.zeros((1, _CH), jnp.int32)
    hi0 = jnp.full((1, _CH), _MAXF_BITS, jnp.int32)
    _, s2, _, st = lax.fori_loop(0, 31, body, (lo0, hi0, lo0, hi0))
    sigma = lax.bitcast_convert_type(s2, jnp.float32)
    vt = lax.bitcast_convert_type(st, jnp.float32)

    rr = dt / sigma
    wt = jnp.where(dt < vt, jnp.exp(-(rr * rr)), 0.0)
    wt2 = wt * wt
    rowsq = jnp.sum(wt2, axis=0, keepdims=True)
    colsq = lax.dot_general(ones_row, wt2, (((1,), (1,)), ((), ())),
                            precision=hi)
    ri = lax.broadcasted_iota(jnp.int32, (_CH, _CH), 0)
    ci = lax.broadcasted_iota(jnp.int32, (_CH, _CH), 1)
    wdiag = jnp.sum(jnp.where(ri == ci, wt, 0.0), axis=0, keepdims=True)
    f2 = colsq + rowsq
    eps = jnp.where(f2 > 0.0, wdiag / jnp.sqrt(jnp.maximum(f2, 1e-38)), 0.0)

    s_ref[...] = alpha * (1.0 + eps)


def _scale_kernel(s_ref, x_ref, o_ref):
    s2 = s_ref[...]
    o_ref[0:3136, :] = x_ref[0:3136, :] * s2[0]
    o_ref[3136:6272, :] = x_ref[3136:6272, :] * s2[1]


def kernel(x):
    b, c, h, w = x.shape
    hw = h * w
    xt = jnp.transpose(x, (0, 2, 3, 1)).reshape(b * hw, c)

    m = pl.pallas_call(
        _max_kernel,
        grid=(b // 4,),
        in_specs=[pl.BlockSpec((4 * hw, c), lambda i: (i, 0))],
        out_specs=pl.BlockSpec((1, 4, c), lambda i: (i, 0, 0)),
        out_shape=jax.ShapeDtypeStruct((b // 4, 4, c), jnp.float32),
    )(xt)

    scale = pl.pallas_call(
        _mid_kernel,
        out_shape=jax.ShapeDtypeStruct((b, c), jnp.float32),
    )(m.reshape(b, c))

    out = pl.pallas_call(
        _scale_kernel,
        grid=(b // 2,),
        in_specs=[
            pl.BlockSpec((2, 1, c), lambda i: (i, 0, 0)),
            pl.BlockSpec((2 * hw, c), lambda i: (i, 0)),
        ],
        out_specs=pl.BlockSpec((2 * hw, c), lambda i: (i, 0)),
        out_shape=jax.ShapeDtypeStruct((b * hw, c), jnp.float32),
    )(scale.reshape(b, 1, c), xt)

    return jnp.transpose(out.reshape(b, h, w, c), (0, 3, 1, 2))
